# baseline (device time: 736069 ns/iter reference)
import jax
import jax.numpy as jnp
from jax import lax
from jax.experimental import pallas as pl
from jax.experimental.pallas import tpu as pltpu

N_DEV = 16
NSTEP = 2 * (N_DEV - 1)


def kernel(x, w_mat):
    m, k_shard = x.shape
    _, n = w_mat.shape
    chunk = m // N_DEV
    h = n // 2
    hh = chunk // 2

    def body(x_ref, w_ref, out_ref, comm_f, comm_b, part_fr, part_br,
             send_f, recv_f, send_b, recv_b, copy_f, copy_b,
             credit_f, credit_b):
        my = lax.axis_index("i")
        left = (my + N_DEV - 1) % N_DEV
        right = (my + 1) % N_DEV

        barrier_sem = pltpu.get_barrier_semaphore()
        for nbr in (left, right):
            pl.semaphore_signal(
                barrier_sem, inc=1,
                device_id=(nbr,), device_id_type=pl.DeviceIdType.MESH,
            )
        pl.semaphore_wait(barrier_sem, 2)

        for q in range(2):
            pl.semaphore_signal(
                credit_f.at[q], inc=1,
                device_id=(left,), device_id_type=pl.DeviceIdType.MESH,
            )
            pl.semaphore_signal(
                credit_b.at[q], inc=1,
                device_id=(right,), device_id_type=pl.DeviceIdType.MESH,
            )

        def pf(c):
            return jnp.dot(
                x_ref[pl.ds(c * chunk, chunk), :],
                w_ref[:, :h],
                preferred_element_type=jnp.float32,
            )

        def pb(c):
            return jnp.dot(
                x_ref[pl.ds(c * chunk, chunk), :],
                w_ref[:, h:],
                preferred_element_type=jnp.float32,
            )

        def mk(ring, s, q):
            ss, rs = s % 2, (s + 1) % 2
            buf, ssem, rsem, dev = (
                (comm_f, send_f, recv_f, right) if ring == "f"
                else (comm_b, send_b, recv_b, left)
            )
            return pltpu.make_async_remote_copy(
                src_ref=buf.at[ss, pl.ds(q * hh, hh), :],
                dst_ref=buf.at[rs, pl.ds(q * hh, hh), :],
                send_sem=ssem.at[ss, q],
                recv_sem=rsem.at[rs, q],
                device_id=(dev,),
                device_id_type=pl.DeviceIdType.MESH,
            )

        def issue(s, q):
            pl.semaphore_wait(credit_f.at[q], 1)
            fd = mk("f", s, q)
            fd.start()
            pl.semaphore_wait(credit_b.at[q], 1)
            bd = mk("b", s, q)
            bd.start()
            return fd, bd

        def grant(q):
            pl.semaphore_signal(
                credit_f.at[q], inc=1,
                device_id=(left,), device_id_type=pl.DeviceIdType.MESH,
            )
            pl.semaphore_signal(
                credit_b.at[q], inc=1,
                device_id=(right,), device_id_type=pl.DeviceIdType.MESH,
            )

        def start_out_copies(slot, cf, cb):
            cpf = pltpu.make_async_copy(
                comm_f.at[slot],
                out_ref.at[pl.ds(cf * chunk, chunk), pl.ds(0, h)],
                copy_f,
            )
            cpf.start()
            cpb = pltpu.make_async_copy(
                comm_b.at[slot],
                out_ref.at[pl.ds(cb * chunk, chunk), pl.ds(h, h)],
                copy_b,
            )
            cpb.start()
            return cpf, cpb

        rA = pl.ds(0, hh)
        rB = pl.ds(hh, hh)

        def phalf(q, lo, width):
            rows = pl.ds(my * chunk + q * hh, hh)
            return jnp.dot(
                x_ref[rows, :], w_ref[:, lo:lo + width],
                preferred_element_type=jnp.float32,
            ).astype(jnp.bfloat16)

        comm_f[0, rA] = phalf(0, 0, h)
        comm_b[0, rA] = phalf(0, h, h)
        curA = issue(0, 0)
        comm_f[0, rB] = phalf(1, 0, h)
        comm_b[0, rB] = phalf(1, h, h)
        curB = issue(0, 1)
        pending = None
        for s in range(NSTEP):
            ss, rs = s % 2, (s + 1) % 2

            if s < N_DEV - 1:
                part_fr[...] = pf((my - 1 - s) % N_DEV).astype(jnp.bfloat16)
                part_br[...] = pb((my + 1 + s) % N_DEV).astype(jnp.bfloat16)
            if s >= N_DEV - 1:
                t_prev = s - 1 - (N_DEV - 1)
                if t_prev < 0:
                    cf_prev = (my + 1) % N_DEV
                    cb_prev = (my - 1) % N_DEV
                else:
                    cf_prev = (my - t_prev) % N_DEV
                    cb_prev = (my + t_prev) % N_DEV
                pending = start_out_copies(ss, cf_prev, cb_prev)

            curA[0].wait()
            curA[1].wait()
            if pending is not None:
                pending[0].wait()
                pending[1].wait()
                pending = None
            if s < NSTEP - 1:
                grant(0)
            if s < N_DEV - 1:
                comm_f[rs, rA] = comm_f[rs, rA] + part_fr[rA]
                comm_b[rs, rA] = comm_b[rs, rA] + part_br[rA]
            if s + 1 < NSTEP:
                nxtA = issue(s + 1, 0)

            curB[0].wait()
            curB[1].wait()
            if s < NSTEP - 1:
                grant(1)
            if s < N_DEV - 1:
                comm_f[rs, rB] = comm_f[rs, rB] + part_fr[rB]
                comm_b[rs, rB] = comm_b[rs, rB] + part_br[rB]
            if s + 1 < NSTEP:
                curA = nxtA
                curB = issue(s + 1, 1)

        last = N_DEV - 2
        pending = start_out_copies(0, (my - last) % N_DEV, (my + last) % N_DEV)
        pending[0].wait()
        pending[1].wait()

    return pl.pallas_call(
        body,
        out_shape=jax.ShapeDtypeStruct((m, n), jnp.bfloat16),
        in_specs=[
            pl.BlockSpec(memory_space=pltpu.VMEM),
            pl.BlockSpec(memory_space=pltpu.VMEM),
        ],
        out_specs=pl.BlockSpec(memory_space=pl.ANY),
        scratch_shapes=[
            pltpu.VMEM((2, chunk, h), jnp.bfloat16),
            pltpu.VMEM((2, chunk, h), jnp.bfloat16),
            pltpu.VMEM((chunk, h), jnp.bfloat16),
            pltpu.VMEM((chunk, h), jnp.bfloat16),
            pltpu.SemaphoreType.DMA((2, 2)),
            pltpu.SemaphoreType.DMA((2, 2)),
            pltpu.SemaphoreType.DMA((2, 2)),
            pltpu.SemaphoreType.DMA((2, 2)),
            pltpu.SemaphoreType.DMA,
            pltpu.SemaphoreType.DMA,
            pltpu.SemaphoreType.REGULAR((2,)),
            pltpu.SemaphoreType.REGULAR((2,)),
        ],
        compiler_params=pltpu.CompilerParams(collective_id=0),
    )(x, w_mat)


# device time: 735995 ns/iter; 1.0001x vs baseline; 1.0001x over previous
import jax
import jax.numpy as jnp
from jax import lax
from jax.experimental import pallas as pl
from jax.experimental.pallas import tpu as pltpu

N_DEV = 16
NSTEP = 2 * (N_DEV - 1)


def kernel(x, w_mat):
    m, k_shard = x.shape
    _, n = w_mat.shape
    chunk = m // N_DEV
    h = n // 2
    hh = chunk // 2

    def body(x_ref, w_ref, out_ref, comm_f, comm_b, part_fr, part_br,
             send_f, recv_f, send_b, recv_b, copy_f, copy_b,
             credit_f, credit_b):
        my = lax.axis_index("i")
        left = (my + N_DEV - 1) % N_DEV
        right = (my + 1) % N_DEV

        barrier_sem = pltpu.get_barrier_semaphore()
        for nbr in (left, right):
            pl.semaphore_signal(
                barrier_sem, inc=1,
                device_id=(nbr,), device_id_type=pl.DeviceIdType.MESH,
            )
        pl.semaphore_wait(barrier_sem, 2)

        for q in range(2):
            pl.semaphore_signal(
                credit_f.at[q], inc=1,
                device_id=(left,), device_id_type=pl.DeviceIdType.MESH,
            )
            pl.semaphore_signal(
                credit_b.at[q], inc=1,
                device_id=(right,), device_id_type=pl.DeviceIdType.MESH,
            )

        def pf(c):
            return jnp.dot(
                x_ref[pl.ds(c * chunk, chunk), :],
                w_ref[:, :h],
                preferred_element_type=jnp.float32,
            )

        def pb(c):
            return jnp.dot(
                x_ref[pl.ds(c * chunk, chunk), :],
                w_ref[:, h:],
                preferred_element_type=jnp.float32,
            )

        def mk(ring, s, q):
            ss, rs = s % 2, (s + 1) % 2
            buf, ssem, rsem, dev = (
                (comm_f, send_f, recv_f, right) if ring == "f"
                else (comm_b, send_b, recv_b, left)
            )
            return pltpu.make_async_remote_copy(
                src_ref=buf.at[ss, pl.ds(q * hh, hh), :],
                dst_ref=buf.at[rs, pl.ds(q * hh, hh), :],
                send_sem=ssem.at[ss, q],
                recv_sem=rsem.at[rs, q],
                device_id=(dev,),
                device_id_type=pl.DeviceIdType.MESH,
            )

        def issue(s, q):
            pl.semaphore_wait(credit_f.at[q], 1)
            fd = mk("f", s, q)
            fd.start()
            pl.semaphore_wait(credit_b.at[q], 1)
            bd = mk("b", s, q)
            bd.start()
            return fd, bd

        def grant(q):
            pl.semaphore_signal(
                credit_f.at[q], inc=1,
                device_id=(left,), device_id_type=pl.DeviceIdType.MESH,
            )
            pl.semaphore_signal(
                credit_b.at[q], inc=1,
                device_id=(right,), device_id_type=pl.DeviceIdType.MESH,
            )

        def start_out_copies(slot, cf, cb):
            cpf = pltpu.make_async_copy(
                comm_f.at[slot],
                out_ref.at[pl.ds(cf * chunk, chunk), pl.ds(0, h)],
                copy_f,
            )
            cpf.start()
            cpb = pltpu.make_async_copy(
                comm_b.at[slot],
                out_ref.at[pl.ds(cb * chunk, chunk), pl.ds(h, h)],
                copy_b,
            )
            cpb.start()
            return cpf, cpb

        rA = pl.ds(0, hh)
        rB = pl.ds(hh, hh)

        comm_f[0] = pf(my).astype(jnp.bfloat16)
        comm_b[0] = pb(my).astype(jnp.bfloat16)

        curA = issue(0, 0)
        curB = issue(0, 1)
        pending = None
        for s in range(NSTEP):
            ss, rs = s % 2, (s + 1) % 2

            if s < N_DEV - 1:
                part_fr[...] = pf((my - 1 - s) % N_DEV).astype(jnp.bfloat16)
                part_br[...] = pb((my + 1 + s) % N_DEV).astype(jnp.bfloat16)
            if s >= N_DEV - 1:
                t_prev = s - 1 - (N_DEV - 1)
                if t_prev < 0:
                    cf_prev = (my + 1) % N_DEV
                    cb_prev = (my - 1) % N_DEV
                else:
                    cf_prev = (my - t_prev) % N_DEV
                    cb_prev = (my + t_prev) % N_DEV
                pending = start_out_copies(ss, cf_prev, cb_prev)

            curA[0].wait()
            curA[1].wait()
            if pending is not None:
                pending[0].wait()
                pending[1].wait()
                pending = None
            if s < NSTEP - 1:
                grant(0)
            if s < N_DEV - 1:
                comm_f[rs, rA] = comm_f[rs, rA] + part_fr[rA]
                comm_b[rs, rA] = comm_b[rs, rA] + part_br[rA]
            if s + 1 < NSTEP:
                nxtA = issue(s + 1, 0)

            curB[0].wait()
            curB[1].wait()
            if s < NSTEP - 1:
                grant(1)
            if s < N_DEV - 1:
                comm_f[rs, rB] = comm_f[rs, rB] + part_fr[rB]
                comm_b[rs, rB] = comm_b[rs, rB] + part_br[rB]
            if s + 1 < NSTEP:
                curA = nxtA
                curB = issue(s + 1, 1)

        last = N_DEV - 2
        pending = start_out_copies(0, (my - last) % N_DEV, (my + last) % N_DEV)
        pending[0].wait()
        pending[1].wait()

    return pl.pallas_call(
        body,
        out_shape=jax.ShapeDtypeStruct((m, n), jnp.bfloat16),
        in_specs=[
            pl.BlockSpec(memory_space=pltpu.VMEM),
            pl.BlockSpec(memory_space=pltpu.VMEM),
        ],
        out_specs=pl.BlockSpec(memory_space=pl.ANY),
        scratch_shapes=[
            pltpu.VMEM((2, chunk, h), jnp.bfloat16),
            pltpu.VMEM((2, chunk, h), jnp.bfloat16),
            pltpu.VMEM((chunk, h), jnp.bfloat16),
            pltpu.VMEM((chunk, h), jnp.bfloat16),
            pltpu.SemaphoreType.DMA((2, 2)),
            pltpu.SemaphoreType.DMA((2, 2)),
            pltpu.SemaphoreType.DMA((2, 2)),
            pltpu.SemaphoreType.DMA((2, 2)),
            pltpu.SemaphoreType.DMA,
            pltpu.SemaphoreType.DMA,
            pltpu.SemaphoreType.REGULAR((2,)),
            pltpu.SemaphoreType.REGULAR((2,)),
        ],
        compiler_params=pltpu.CompilerParams(collective_id=0),
    )(x, w_mat)
